# baseline (speedup 1.0000x reference)
import functools

import jax
import jax.numpy as jnp
import numpy as np
from jax import lax
from jax.experimental import pallas as pl
from jax.experimental.pallas import tpu as pltpu
from jax.experimental.pallas import tpu_sc as plsc

N = 10000
E = 160000
H = 64
CUTOFF_UPPER = 5.0

BN = 400
BE = 1280
BN2 = 400


def _node_prep_body(x_ref, wt0_ref, wt1_ref, wt2_ref, xn_ref, u_ref):
    x = x_ref[...]
    nrm = jnp.sum(x * x, axis=(1, 2))
    xn = x * (1.0 / (nrm + 1.0))[:, None, None, :]
    xn_ref[...] = xn
    p = [xn[:, k // 3, k % 3, :] for k in range(9)]
    I = (p[0] + p[4] + p[8]) * (1.0 / 3.0)
    a01 = 0.5 * (p[1] - p[3])
    a02 = 0.5 * (p[2] - p[6])
    a12 = 0.5 * (p[5] - p[7])
    s00 = p[0] - I
    s11 = p[4] - I
    s01 = 0.5 * (p[1] + p[3])
    s02 = 0.5 * (p[2] + p[6])
    s12 = 0.5 * (p[5] + p[7])
    wt0 = wt0_ref[...]
    wt1 = wt1_ref[...]
    wt2 = wt2_ref[...]
    dot = functools.partial(jnp.dot, preferred_element_type=jnp.float32)
    Iw = dot(I, wt0)
    a01w, a02w, a12w = dot(a01, wt1), dot(a02, wt1), dot(a12, wt1)
    s00w, s01w, s02w = dot(s00, wt2), dot(s01, wt2), dot(s02, wt2)
    s11w, s12w = dot(s11, wt2), dot(s12, wt2)
    for k, v in enumerate([Iw, a01w, a02w, a12w, s00w, s01w, s02w, s11w, s12w]):
        u_ref[k] = v


def _node_prep(Xr, Wt0, Wt1, Wt2):
    full = lambda shp: pl.BlockSpec(shp, lambda i: (0,) * len(shp))
    return pl.pallas_call(
        _node_prep_body,
        grid=(N // BN,),
        in_specs=[
            pl.BlockSpec((BN, 3, 3, 64), lambda i: (i, 0, 0, 0)),
            full((64, 64)), full((64, 64)), full((64, 64)),
        ],
        out_specs=[
            pl.BlockSpec((BN, 3, 3, 64), lambda i: (i, 0, 0, 0)),
            pl.BlockSpec((9, BN, 64), lambda i: (0, i, 0)),
        ],
        out_shape=[
            jax.ShapeDtypeStruct((N, 3, 3, 64), jnp.float32),
            jax.ShapeDtypeStruct((9, N, 64), jnp.float32),
        ],
    )(Xr, Wt0, Wt1, Wt2)


def _edge_mlp_body(ea_ref, ci_ref, cj_ref, ew_ref,
                   w0a_ref, w0b_ref, w0c_ref, b0_ref,
                   w1_ref, b1_ref, w2_ref, b2_ref, xw_ref):
    dot = functools.partial(jnp.dot, preferred_element_type=jnp.float32)
    bf = jnp.bfloat16
    h = (dot(ea_ref[...].astype(bf), w0a_ref[...])
         + dot(ci_ref[...].astype(bf), w0b_ref[...])
         + dot(cj_ref[...].astype(bf), w0c_ref[...]) + b0_ref[...])
    h = jax.nn.silu(h.astype(bf))
    h = jax.nn.silu((dot(h, w1_ref[...]) + b1_ref[...]).astype(bf))
    h = jax.nn.silu((dot(h, w2_ref[...]) + b2_ref[...]).astype(bf))
    d = ew_ref[0]
    c = 0.5 * (jnp.cos(d * (jnp.pi / CUTOFF_UPPER)) + 1.0)
    c = jnp.where(d < CUTOFF_UPPER, c, 0.0)
    h3 = h.astype(jnp.float32).reshape(BE // 128, 128, 192) * c[:, :, None]
    h = h3.reshape(BE, 192).astype(bf)
    xw_ref[0] = h[:, 0:64]
    xw_ref[1] = h[:, 64:128]
    xw_ref[2] = h[:, 128:192]


def _edge_mlp(ea, ci, cj, ew2, W0a, W0b, W0c, b0, W1, b1, W2, b2):
    full = lambda shp: pl.BlockSpec(shp, lambda i: (0,) * len(shp))
    return pl.pallas_call(
        _edge_mlp_body,
        grid=(E // BE,),
        in_specs=[
            pl.BlockSpec((BE, 64), lambda i: (i, 0)),
            pl.BlockSpec((BE, 16), lambda i: (i, 0)),
            pl.BlockSpec((BE, 16), lambda i: (i, 0)),
            pl.BlockSpec((1, BE // 128, 128), lambda i: (i, 0, 0)),
            full((64, 64)), full((16, 64)), full((16, 64)), full((1, 64)),
            full((64, 128)), full((1, 128)), full((128, 192)), full((1, 192)),
        ],
        out_specs=pl.BlockSpec((3, BE, 64), lambda i: (0, i, 0)),
        out_shape=jax.ShapeDtypeStruct((3, E, 64), jnp.bfloat16),
    )(ea, ci, cj, ew2, W0a, W0b, W0c, b0, W1, b1, W2, b2)


def _entries_from_planes(I, a01, a02, a12, s00, s01, s02, s11, s12):
    return [
        I + s00, a01 + s01, a02 + s02,
        -a01 + s01, I + s11, a12 + s12,
        -a02 + s02, -a12 + s12, I - s00 - s11,
    ]


def _epilogue_body(xn_ref, u_ref, m_ref, wt3_ref, wt4_ref, wt5_ref, out_ref):
    u = u_ref[...]
    m = m_ref[...]
    yp = [u[k] for k in range(9)]
    mp = [m[k] + m[9 + k] for k in range(9)]
    y = _entries_from_planes(*yp)
    mm = _entries_from_planes(*mp)
    c = [None] * 9
    for i in range(3):
        for j in range(3):
            acc = 0.0
            for k in range(3):
                acc = acc + y[3 * i + k] * mm[3 * k + j] + mm[3 * i + k] * y[3 * k + j]
            c[3 * i + j] = acc
    I2 = (c[0] + c[4] + c[8]) * (1.0 / 3.0)
    a01 = 0.5 * (c[1] - c[3])
    a02 = 0.5 * (c[2] - c[6])
    a12 = 0.5 * (c[5] - c[7])
    s00 = c[0] - I2
    s11 = c[4] - I2
    s01 = 0.5 * (c[1] + c[3])
    s02 = 0.5 * (c[2] + c[6])
    s12 = 0.5 * (c[5] + c[7])
    nrm = c[0] * c[0]
    for k in range(1, 9):
        nrm = nrm + c[k] * c[k]
    inv = 1.0 / (nrm + 1.0)
    dot = functools.partial(jnp.dot, preferred_element_type=jnp.float32)
    wt3, wt4, wt5 = wt3_ref[...], wt4_ref[...], wt5_ref[...]
    I3 = dot(I2 * inv, wt3)
    a01, a02, a12 = dot(a01 * inv, wt4), dot(a02 * inv, wt4), dot(a12 * inv, wt4)
    s00, s01, s02 = dot(s00 * inv, wt5), dot(s01 * inv, wt5), dot(s02 * inv, wt5)
    s11, s12 = dot(s11 * inv, wt5), dot(s12 * inv, wt5)
    d = _entries_from_planes(I3, a01, a02, a12, s00, s01, s02, s11, s12)
    xn = xn_ref[...]
    outs = []
    for i in range(3):
        for j in range(3):
            dd = 0.0
            for k in range(3):
                dd = dd + d[3 * i + k] * d[3 * k + j]
            outs.append((xn[:, i, j, :] + d[3 * i + j] + dd)[:, None, :])
    out_ref[...] = jnp.concatenate(outs, axis=1).reshape(BN2, 3, 3, 64)


def _epilogue(Xn, U, M, Wt3, Wt4, Wt5):
    full = lambda shp: pl.BlockSpec(shp, lambda i: (0,) * len(shp))
    return pl.pallas_call(
        _epilogue_body,
        grid=(N // BN2,),
        in_specs=[
            pl.BlockSpec((BN2, 3, 3, 64), lambda i: (i, 0, 0, 0)),
            pl.BlockSpec((9, BN2, 64), lambda i: (0, i, 0)),
            pl.BlockSpec((18, BN2, 64), lambda i: (0, i, 0)),
            full((64, 64)), full((64, 64)), full((64, 64)),
        ],
        out_specs=pl.BlockSpec((BN2, 3, 3, 64), lambda i: (i, 0, 0, 0)),
        out_shape=jax.ShapeDtypeStruct((N, 3, 3, 64), jnp.float32),
    )(Xn, U, M, Wt3, Wt4, Wt5)


_GB = 128
_NBATCH = E // _GB
_MESH = plsc.VectorSubcoreMesh(core_axis_name="c", subcore_axis_name="s",
                               num_cores=2, num_subcores=16)


def _gather_charges_body(ch_ref, dst_ref, src_ref, ci_ref, cj_ref,
                         dix, six, rowsa, rowsb, sema, semb):
    c = lax.axis_index("c")
    s = lax.axis_index("s")
    w = s * 2 + c

    def issue(j, slot):
        @pl.when(w + j * 32 < _NBATCH)
        def _():
            off = (w + j * 32) * _GB
            pltpu.sync_copy(dst_ref.at[pl.ds(off, _GB)], dix.at[slot])
            pltpu.sync_copy(src_ref.at[pl.ds(off, _GB)], six.at[slot])
            pltpu.async_copy(ch_ref.at[dix.at[slot]], rowsa.at[slot], sema.at[slot])
            pltpu.async_copy(ch_ref.at[six.at[slot]], rowsb.at[slot], semb.at[slot])

    def finish(j, slot):
        @pl.when(w + j * 32 < _NBATCH)
        def _():
            off = (w + j * 32) * _GB
            pltpu.make_async_copy(ch_ref.at[dix.at[slot]], rowsa.at[slot],
                                  sema.at[slot]).wait()
            pltpu.sync_copy(rowsa.at[slot], ci_ref.at[pl.ds(off, _GB)])
            pltpu.make_async_copy(ch_ref.at[six.at[slot]], rowsb.at[slot],
                                  semb.at[slot]).wait()
            pltpu.sync_copy(rowsb.at[slot], cj_ref.at[pl.ds(off, _GB)])

    issue(0, 0)
    issue(1, 1)

    def body(jj, _):
        for b01 in range(2):
            j = jj * 2 + b01
            finish(j, b01)
            issue(j + 2, b01)
        return 0
    lax.fori_loop(0, (_NBATCH + 63) // 64, body, 0)


def _gather_charges_sc(charges16, dst, src):
    f = pl.kernel(
        _gather_charges_body,
        out_type=[jax.ShapeDtypeStruct((E, 16), jnp.float32),
                  jax.ShapeDtypeStruct((E, 16), jnp.float32)],
        mesh=_MESH,
        compiler_params=pltpu.CompilerParams(use_tc_tiling_on_sc=False),
        scratch_types=[
            pltpu.VMEM((2, _GB), jnp.int32),
            pltpu.VMEM((2, _GB), jnp.int32),
            pltpu.VMEM((2, _GB, 16), jnp.float32),
            pltpu.VMEM((2, _GB, 16), jnp.float32),
            pltpu.SemaphoreType.DMA((2,)),
            pltpu.SemaphoreType.DMA((2,)),
        ],
    )
    return f(charges16, dst, src)


_MB = 128
_EC = E // 2
_MNB = _EC // _MB
_RPT = N // 16
_WSEL = (0, 1, 1, 1, 2, 2, 2, 2, 2)


def _messages_body(u_ref, xw_ref, src_ref, dst_ref, m_ref,
                   acc, ubuf, wbuf, mbuf, sidx, didx, zbuf,
                   semg, semw, sems, semsi):
    c = lax.axis_index("c")
    t = lax.axis_index("s")
    zv = jnp.zeros((16,), jnp.float32)

    def zrow(r, _):
        for k in range(4):
            zbuf[r, pl.ds(k * 16, 16)] = zv
        return 0
    lax.fori_loop(0, 125, zrow, 0)

    def make_pass(p):
        w = _WSEL[p]
        up = u_ref.at[p]

        def guard(j, fn):
            @pl.when(t + j * 16 < _MNB)
            def _():
                fn()

        def eoff_of(j):
            return c * _EC + (t + j * 16) * _MB

        def stage_i(j, s4):
            def fn():
                pltpu.async_copy(src_ref.at[pl.ds(eoff_of(j), _MB)],
                                 sidx.at[s4], semsi.at[s4])
            guard(j, fn)

        def eoff2_of(j):
            return c * (_EC // 2) + (t + j * 16) * (_MB // 2)

        def stage_g(j, s2, s4):
            def fn():
                pltpu.make_async_copy(src_ref.at[pl.ds(eoff_of(j), _MB)],
                                      sidx.at[s4], semsi.at[s4]).wait()
                pltpu.async_copy(up.at[sidx.at[s4]], ubuf.at[s2], semg.at[s2])
                pltpu.async_copy(xw_ref.at[w, pl.ds(eoff_of(j), _MB)],
                                 wbuf.at[s2], semw.at[s2])
            guard(j, fn)

        def stage_c(j, s2, s4):
            def fn():
                pltpu.make_async_copy(up.at[sidx.at[s4]], ubuf.at[s2],
                                      semg.at[s2]).wait()
                pltpu.make_async_copy(xw_ref.at[w, pl.ds(eoff_of(j), _MB)],
                                      wbuf.at[s2], semw.at[s2]).wait()
                pltpu.sync_copy(dst_ref.at[pl.ds(eoff_of(j), _MB)], didx.at[s4])

                @pl.when(j >= 2)
                def _():
                    pltpu.make_async_copy(mbuf.at[s2], acc.at[didx.at[s4]],
                                          sems.at[s2]).wait()

                def edge(e, _):
                    for k in range(2):
                        wv = wbuf[s2, e, pl.ds(k * 32, 32)]
                        wa, wb2 = plsc.unpack(
                            wv, format=plsc.PackFormat.INTERLEAVED)
                        mbuf[s2, e, pl.ds(k * 32, 16)] = (
                            ubuf[s2, e, pl.ds(k * 32, 16)] * wa)
                        mbuf[s2, e, pl.ds(k * 32 + 16, 16)] = (
                            ubuf[s2, e, pl.ds(k * 32 + 16, 16)] * wb2)
                    return 0
                lax.fori_loop(0, _MB, edge, 0)
                pltpu.async_copy(mbuf.at[s2], acc.at[didx.at[s4]],
                                 sems.at[s2], add=True)
            guard(j, fn)

        for piece in range(5):
            pltpu.sync_copy(zbuf, acc.at[pl.ds(t * _RPT + piece * 125, 125)])
        plsc.subcore_barrier()

        for k in range(4):
            stage_i(k, k)
        stage_g(0, 0, 0)
        stage_g(1, 1, 1)

        def body(jj, _):
            for k in range(4):
                j = jj * 4 + k
                stage_c(j, k % 2, k)
                stage_g(j + 2, k % 2, (k + 2) % 4)
                stage_i(j + 4, k)
            return 0
        lax.fori_loop(0, 10, body, 0)
        for k in range(2):
            j = 38 + k

            @pl.when(t + j * 16 < _MNB)
            def _(j=j, k=k):
                pltpu.make_async_copy(mbuf.at[k], acc.at[didx.at[j % 4]],
                                      sems.at[k]).wait()
        plsc.subcore_barrier()
        for cc in range(2):
            @pl.when(c == cc)
            def _(cc=cc):
                pltpu.sync_copy(acc.at[pl.ds(t * _RPT, _RPT)],
                                m_ref.at[cc * 9 + p, pl.ds(t * _RPT, _RPT)])
        plsc.subcore_barrier()

    for p in range(9):
        make_pass(p)


def _messages_sc(U, xw, src, dst):
    f = pl.kernel(
        _messages_body,
        out_type=jax.ShapeDtypeStruct((18, N, 64), jnp.float32),
        mesh=_MESH,
        compiler_params=pltpu.CompilerParams(use_tc_tiling_on_sc=False,
                                             needs_layout_passes=False),
        scratch_types=[
            pltpu.VMEM_SHARED((N, 64), jnp.float32),
            pltpu.VMEM((2, _MB, 64), jnp.float32),
            pltpu.VMEM((2, _MB, 64), jnp.bfloat16),
            pltpu.VMEM((2, _MB, 64), jnp.float32),
            pltpu.VMEM((4, _MB), jnp.int32),
            pltpu.VMEM((4, _MB), jnp.int32),
            pltpu.VMEM((125, 64), jnp.float32),
            pltpu.SemaphoreType.DMA((2,)),
            pltpu.SemaphoreType.DMA((2,)),
            pltpu.SemaphoreType.DMA((2,)),
            pltpu.SemaphoreType.DMA((4,)),
        ],
    )
    return f(U, xw, src, dst)


def kernel(X, charges, edge_index, edge_weight, edge_attr,
           W0, b0, W1, b1, W2, b2, Wt0, Wt1, Wt2, Wt3, Wt4, Wt5):
    dst = edge_index[0]
    src = edge_index[1]
    charges16 = jnp.pad(charges, ((0, 0), (0, 8)))
    bf = jnp.bfloat16
    W0a = W0[:64].astype(bf)
    W0b = jnp.pad(W0[64:72], ((0, 8), (0, 0))).astype(bf)
    W0c = jnp.pad(W0[72:80], ((0, 8), (0, 0))).astype(bf)
    ew2 = edge_weight.reshape(E // BE, BE // 128, 128)

    idx64 = np.empty(64, np.int32)
    for blk in range(2):
        for i in range(16):
            idx64[blk * 32 + 2 * i] = blk * 32 + i
            idx64[blk * 32 + 2 * i + 1] = blk * 32 + 16 + i
    perm192 = np.concatenate([idx64 + 64 * p for p in range(3)])
    W2p = W2[:, perm192]
    b2p = b2[perm192]

    Xn, U = _node_prep(X, Wt0, Wt1, Wt2)
    ci, cj = _gather_charges_sc(charges16, dst, src)
    xw = _edge_mlp(edge_attr, ci, cj, ew2,
                   W0a, W0b, W0c, b0[None, :], W1.astype(bf), b1[None, :],
                   W2p.astype(bf), b2p[None, :])
    M = _messages_sc(U, xw, src, dst)
    return _epilogue(Xn, U, M, Wt3, Wt4, Wt5)


# speedup vs baseline: 1.2194x; 1.2194x over previous
import functools

import jax
import jax.numpy as jnp
import numpy as np
from jax import lax
from jax.experimental import pallas as pl
from jax.experimental.pallas import tpu as pltpu
from jax.experimental.pallas import tpu_sc as plsc

N = 10000
E = 160000
H = 64
CUTOFF_UPPER = 5.0

BN = 400
BE = 1280
BN2 = 400


def _node_prep_body(x_ref, wt0_ref, wt1_ref, wt2_ref, xn_ref, u_ref):
    x = x_ref[...]
    nrm = jnp.sum(x * x, axis=(1, 2))
    xn = x * (1.0 / (nrm + 1.0))[:, None, None, :]
    xn_ref[...] = xn
    p = [xn[:, k // 3, k % 3, :] for k in range(9)]
    I = (p[0] + p[4] + p[8]) * (1.0 / 3.0)
    a01 = 0.5 * (p[1] - p[3])
    a02 = 0.5 * (p[2] - p[6])
    a12 = 0.5 * (p[5] - p[7])
    s00 = p[0] - I
    s11 = p[4] - I
    s01 = 0.5 * (p[1] + p[3])
    s02 = 0.5 * (p[2] + p[6])
    s12 = 0.5 * (p[5] + p[7])
    wt0 = wt0_ref[...]
    wt1 = wt1_ref[...]
    wt2 = wt2_ref[...]
    dot = functools.partial(jnp.dot, preferred_element_type=jnp.float32)
    Iw = dot(I, wt0)
    a01w, a02w, a12w = dot(a01, wt1), dot(a02, wt1), dot(a12, wt1)
    s00w, s01w, s02w = dot(s00, wt2), dot(s01, wt2), dot(s02, wt2)
    s11w, s12w = dot(s11, wt2), dot(s12, wt2)
    for k, v in enumerate([Iw, a01w, a02w, a12w, s00w, s01w, s02w, s11w, s12w]):
        u_ref[k] = v


def _node_prep(Xr, Wt0, Wt1, Wt2):
    full = lambda shp: pl.BlockSpec(shp, lambda i: (0,) * len(shp))
    return pl.pallas_call(
        _node_prep_body,
        grid=(N // BN,),
        in_specs=[
            pl.BlockSpec((BN, 3, 3, 64), lambda i: (i, 0, 0, 0)),
            full((64, 64)), full((64, 64)), full((64, 64)),
        ],
        out_specs=[
            pl.BlockSpec((BN, 3, 3, 64), lambda i: (i, 0, 0, 0)),
            pl.BlockSpec((9, BN, 64), lambda i: (0, i, 0)),
        ],
        out_shape=[
            jax.ShapeDtypeStruct((N, 3, 3, 64), jnp.float32),
            jax.ShapeDtypeStruct((9, N, 64), jnp.float32),
        ],
    )(Xr, Wt0, Wt1, Wt2)


def _edge_mlp_body(ea_ref, ci_ref, cj_ref, ew_ref,
                   w0a_ref, w0b_ref, w0c_ref, b0_ref,
                   w1_ref, b1_ref, w2_ref, b2_ref, xw_ref):
    dot = functools.partial(jnp.dot, preferred_element_type=jnp.float32)
    bf = jnp.bfloat16
    h = (dot(ea_ref[...].astype(bf), w0a_ref[...])
         + dot(ci_ref[...].astype(bf), w0b_ref[...])
         + dot(cj_ref[...].astype(bf), w0c_ref[...]) + b0_ref[...])
    h = jax.nn.silu(h.astype(bf))
    h = jax.nn.silu((dot(h, w1_ref[...]) + b1_ref[...]).astype(bf))
    h = jax.nn.silu((dot(h, w2_ref[...]) + b2_ref[...]).astype(bf))
    d = ew_ref[0]
    c = 0.5 * (jnp.cos(d * (jnp.pi / CUTOFF_UPPER)) + 1.0)
    c = jnp.where(d < CUTOFF_UPPER, c, 0.0)
    h3 = h.astype(jnp.float32).reshape(BE // 128, 128, 192) * c[:, :, None]
    h = h3.reshape(BE, 192)
    xw_ref[0] = h[:, 0:64]
    xw_ref[1] = h[:, 64:128]
    xw_ref[2] = h[:, 128:192]


def _edge_mlp(ea, ci, cj, ew2, W0a, W0b, W0c, b0, W1, b1, W2, b2):
    full = lambda shp: pl.BlockSpec(shp, lambda i: (0,) * len(shp))
    return pl.pallas_call(
        _edge_mlp_body,
        grid=(E // BE,),
        in_specs=[
            pl.BlockSpec((BE, 64), lambda i: (i, 0)),
            pl.BlockSpec((BE, 16), lambda i: (i, 0)),
            pl.BlockSpec((BE, 16), lambda i: (i, 0)),
            pl.BlockSpec((1, BE // 128, 128), lambda i: (i, 0, 0)),
            full((64, 64)), full((16, 64)), full((16, 64)), full((1, 64)),
            full((64, 128)), full((1, 128)), full((128, 192)), full((1, 192)),
        ],
        out_specs=pl.BlockSpec((3, BE, 64), lambda i: (0, i, 0)),
        out_shape=jax.ShapeDtypeStruct((3, E, 64), jnp.float32),
    )(ea, ci, cj, ew2, W0a, W0b, W0c, b0, W1, b1, W2, b2)


def _entries_from_planes(I, a01, a02, a12, s00, s01, s02, s11, s12):
    return [
        I + s00, a01 + s01, a02 + s02,
        -a01 + s01, I + s11, a12 + s12,
        -a02 + s02, -a12 + s12, I - s00 - s11,
    ]


def _epilogue_body(xn_ref, u_ref, m_ref, wt3_ref, wt4_ref, wt5_ref, out_ref):
    u = u_ref[...]
    m = m_ref[...]
    yp = [u[k] for k in range(9)]
    mp = [m[k] + m[9 + k] for k in range(9)]
    y = _entries_from_planes(*yp)
    mm = _entries_from_planes(*mp)
    c = [None] * 9
    for i in range(3):
        for j in range(3):
            acc = 0.0
            for k in range(3):
                acc = acc + y[3 * i + k] * mm[3 * k + j] + mm[3 * i + k] * y[3 * k + j]
            c[3 * i + j] = acc
    I2 = (c[0] + c[4] + c[8]) * (1.0 / 3.0)
    a01 = 0.5 * (c[1] - c[3])
    a02 = 0.5 * (c[2] - c[6])
    a12 = 0.5 * (c[5] - c[7])
    s00 = c[0] - I2
    s11 = c[4] - I2
    s01 = 0.5 * (c[1] + c[3])
    s02 = 0.5 * (c[2] + c[6])
    s12 = 0.5 * (c[5] + c[7])
    nrm = c[0] * c[0]
    for k in range(1, 9):
        nrm = nrm + c[k] * c[k]
    inv = 1.0 / (nrm + 1.0)
    dot = functools.partial(jnp.dot, preferred_element_type=jnp.float32)
    wt3, wt4, wt5 = wt3_ref[...], wt4_ref[...], wt5_ref[...]
    I3 = dot(I2 * inv, wt3)
    a01, a02, a12 = dot(a01 * inv, wt4), dot(a02 * inv, wt4), dot(a12 * inv, wt4)
    s00, s01, s02 = dot(s00 * inv, wt5), dot(s01 * inv, wt5), dot(s02 * inv, wt5)
    s11, s12 = dot(s11 * inv, wt5), dot(s12 * inv, wt5)
    d = _entries_from_planes(I3, a01, a02, a12, s00, s01, s02, s11, s12)
    xn = xn_ref[...]
    outs = []
    for i in range(3):
        for j in range(3):
            dd = 0.0
            for k in range(3):
                dd = dd + d[3 * i + k] * d[3 * k + j]
            outs.append((xn[:, i, j, :] + d[3 * i + j] + dd)[:, None, :])
    out_ref[...] = jnp.concatenate(outs, axis=1).reshape(BN2, 3, 3, 64)


def _epilogue(Xn, U, M, Wt3, Wt4, Wt5):
    full = lambda shp: pl.BlockSpec(shp, lambda i: (0,) * len(shp))
    return pl.pallas_call(
        _epilogue_body,
        grid=(N // BN2,),
        in_specs=[
            pl.BlockSpec((BN2, 3, 3, 64), lambda i: (i, 0, 0, 0)),
            pl.BlockSpec((9, BN2, 64), lambda i: (0, i, 0)),
            pl.BlockSpec((18, BN2, 64), lambda i: (0, i, 0)),
            full((64, 64)), full((64, 64)), full((64, 64)),
        ],
        out_specs=pl.BlockSpec((BN2, 3, 3, 64), lambda i: (i, 0, 0, 0)),
        out_shape=jax.ShapeDtypeStruct((N, 3, 3, 64), jnp.float32),
    )(Xn, U, M, Wt3, Wt4, Wt5)


_GB = 128
_NBATCH = E // _GB
_MESH = plsc.VectorSubcoreMesh(core_axis_name="c", subcore_axis_name="s",
                               num_cores=2, num_subcores=16)


def _gather_charges_body(ch_ref, dst_ref, src_ref, ci_ref, cj_ref,
                         dix, six, rowsa, rowsb, sema, semb):
    c = lax.axis_index("c")
    s = lax.axis_index("s")
    w = s * 2 + c

    def issue(j, slot):
        @pl.when(w + j * 32 < _NBATCH)
        def _():
            off = (w + j * 32) * _GB
            pltpu.sync_copy(dst_ref.at[pl.ds(off, _GB)], dix.at[slot])
            pltpu.sync_copy(src_ref.at[pl.ds(off, _GB)], six.at[slot])
            pltpu.async_copy(ch_ref.at[dix.at[slot]], rowsa.at[slot], sema.at[slot])
            pltpu.async_copy(ch_ref.at[six.at[slot]], rowsb.at[slot], semb.at[slot])

    def finish(j, slot):
        @pl.when(w + j * 32 < _NBATCH)
        def _():
            off = (w + j * 32) * _GB
            pltpu.make_async_copy(ch_ref.at[dix.at[slot]], rowsa.at[slot],
                                  sema.at[slot]).wait()
            pltpu.sync_copy(rowsa.at[slot], ci_ref.at[pl.ds(off, _GB)])
            pltpu.make_async_copy(ch_ref.at[six.at[slot]], rowsb.at[slot],
                                  semb.at[slot]).wait()
            pltpu.sync_copy(rowsb.at[slot], cj_ref.at[pl.ds(off, _GB)])

    issue(0, 0)
    issue(1, 1)

    def body(jj, _):
        for b01 in range(2):
            j = jj * 2 + b01
            finish(j, b01)
            issue(j + 2, b01)
        return 0
    lax.fori_loop(0, (_NBATCH + 63) // 64, body, 0)


def _gather_charges_sc(charges16, dst, src):
    f = pl.kernel(
        _gather_charges_body,
        out_type=[jax.ShapeDtypeStruct((E, 16), jnp.float32),
                  jax.ShapeDtypeStruct((E, 16), jnp.float32)],
        mesh=_MESH,
        compiler_params=pltpu.CompilerParams(use_tc_tiling_on_sc=False),
        scratch_types=[
            pltpu.VMEM((2, _GB), jnp.int32),
            pltpu.VMEM((2, _GB), jnp.int32),
            pltpu.VMEM((2, _GB, 16), jnp.float32),
            pltpu.VMEM((2, _GB, 16), jnp.float32),
            pltpu.SemaphoreType.DMA((2,)),
            pltpu.SemaphoreType.DMA((2,)),
        ],
    )
    return f(charges16, dst, src)


_MB = 128
_EC = E // 2
_MNB = _EC // _MB
_RPT = N // 16
_WSEL = (0, 1, 1, 1, 2, 2, 2, 2, 2)


def _messages_body(u_ref, xw_ref, src_ref, dst_ref, m_ref,
                   acc, ubuf, wbuf, mbuf, sidx, didx, zbuf,
                   semg, semw, sems, semsi):
    c = lax.axis_index("c")
    t = lax.axis_index("s")
    zv = jnp.zeros((16,), jnp.float32)

    def zrow(r, _):
        for k in range(4):
            zbuf[r, pl.ds(k * 16, 16)] = zv
        return 0
    lax.fori_loop(0, 125, zrow, 0)

    def make_pass(p):
        w = _WSEL[p]
        up = u_ref.at[p]

        def guard(j, fn):
            @pl.when(t + j * 16 < _MNB)
            def _():
                fn()

        def eoff_of(j):
            return c * _EC + (t + j * 16) * _MB

        def stage_i(j, s4):
            def fn():
                pltpu.async_copy(src_ref.at[pl.ds(eoff_of(j), _MB)],
                                 sidx.at[s4], semsi.at[s4])
            guard(j, fn)

        def eoff2_of(j):
            return c * (_EC // 2) + (t + j * 16) * (_MB // 2)

        def stage_g(j, s2, s4):
            def fn():
                pltpu.make_async_copy(src_ref.at[pl.ds(eoff_of(j), _MB)],
                                      sidx.at[s4], semsi.at[s4]).wait()
                pltpu.async_copy(up.at[sidx.at[s4]], ubuf.at[s2], semg.at[s2])
                pltpu.async_copy(xw_ref.at[w, pl.ds(eoff_of(j), _MB)],
                                 wbuf.at[s2], semw.at[s2])
            guard(j, fn)

        def stage_c(j, s2, s4):
            def fn():
                pltpu.make_async_copy(up.at[sidx.at[s4]], ubuf.at[s2],
                                      semg.at[s2]).wait()
                pltpu.make_async_copy(xw_ref.at[w, pl.ds(eoff_of(j), _MB)],
                                      wbuf.at[s2], semw.at[s2]).wait()
                pltpu.sync_copy(dst_ref.at[pl.ds(eoff_of(j), _MB)], didx.at[s4])

                @pl.when(j >= 2)
                def _():
                    pltpu.make_async_copy(mbuf.at[s2], acc.at[didx.at[s4]],
                                          sems.at[s2]).wait()

                def edge(e, _):
                    for q in range(4):
                        mbuf[s2, e, pl.ds(q * 16, 16)] = (
                            ubuf[s2, e, pl.ds(q * 16, 16)]
                            * wbuf[s2, e, pl.ds(q * 16, 16)])
                    return 0
                lax.fori_loop(0, _MB, edge, 0)
                pltpu.async_copy(mbuf.at[s2], acc.at[didx.at[s4]],
                                 sems.at[s2], add=True)
            guard(j, fn)

        for piece in range(5):
            pltpu.sync_copy(zbuf, acc.at[pl.ds(t * _RPT + piece * 125, 125)])
        plsc.subcore_barrier()

        for k in range(4):
            stage_i(k, k)
        stage_g(0, 0, 0)
        stage_g(1, 1, 1)

        def body(jj, _):
            for k in range(4):
                j = jj * 4 + k
                stage_c(j, k % 2, k)
                stage_g(j + 2, k % 2, (k + 2) % 4)
                stage_i(j + 4, k)
            return 0
        lax.fori_loop(0, 10, body, 0)
        for k in range(2):
            j = 38 + k

            @pl.when(t + j * 16 < _MNB)
            def _(j=j, k=k):
                pltpu.make_async_copy(mbuf.at[k], acc.at[didx.at[j % 4]],
                                      sems.at[k]).wait()
        plsc.subcore_barrier()
        for cc in range(2):
            @pl.when(c == cc)
            def _(cc=cc):
                pltpu.sync_copy(acc.at[pl.ds(t * _RPT, _RPT)],
                                m_ref.at[cc * 9 + p, pl.ds(t * _RPT, _RPT)])
        plsc.subcore_barrier()

    for p in range(9):
        make_pass(p)


def _messages_sc(U, xw, src, dst):
    f = pl.kernel(
        _messages_body,
        out_type=jax.ShapeDtypeStruct((18, N, 64), jnp.float32),
        mesh=_MESH,
        compiler_params=pltpu.CompilerParams(use_tc_tiling_on_sc=False),
        scratch_types=[
            pltpu.VMEM_SHARED((N, 64), jnp.float32),
            pltpu.VMEM((2, _MB, 64), jnp.float32),
            pltpu.VMEM((2, _MB, 64), jnp.float32),
            pltpu.VMEM((2, _MB, 64), jnp.float32),
            pltpu.VMEM((4, _MB), jnp.int32),
            pltpu.VMEM((4, _MB), jnp.int32),
            pltpu.VMEM((125, 64), jnp.float32),
            pltpu.SemaphoreType.DMA((2,)),
            pltpu.SemaphoreType.DMA((2,)),
            pltpu.SemaphoreType.DMA((2,)),
            pltpu.SemaphoreType.DMA((4,)),
        ],
    )
    return f(U, xw, src, dst)


def kernel(X, charges, edge_index, edge_weight, edge_attr,
           W0, b0, W1, b1, W2, b2, Wt0, Wt1, Wt2, Wt3, Wt4, Wt5):
    dst = edge_index[0]
    src = edge_index[1]
    charges16 = jnp.pad(charges, ((0, 0), (0, 8)))
    bf = jnp.bfloat16
    W0a = W0[:64].astype(bf)
    W0b = jnp.pad(W0[64:72], ((0, 8), (0, 0))).astype(bf)
    W0c = jnp.pad(W0[72:80], ((0, 8), (0, 0))).astype(bf)
    ew2 = edge_weight.reshape(E // BE, BE // 128, 128)

    Xn, U = _node_prep(X, Wt0, Wt1, Wt2)
    ci, cj = _gather_charges_sc(charges16, dst, src)
    xw = _edge_mlp(edge_attr, ci, cj, ew2,
                   W0a, W0b, W0c, b0[None, :], W1.astype(bf), b1[None, :],
                   W2.astype(bf), b2[None, :])
    M = _messages_sc(U, xw, src, dst)
    return _epilogue(Xn, U, M, Wt3, Wt4, Wt5)


# speedup vs baseline: 1.2454x; 1.0213x over previous
import functools

import jax
import jax.numpy as jnp
import numpy as np
from jax import lax
from jax.experimental import pallas as pl
from jax.experimental.pallas import tpu as pltpu
from jax.experimental.pallas import tpu_sc as plsc

N = 10000
E = 160000
H = 64
CUTOFF_UPPER = 5.0

BN = 400
BE = 1280
BN2 = 400


def _node_prep_body(x_ref, wt0_ref, wt1_ref, wt2_ref, xn_ref, u_ref):
    x = x_ref[...]
    nrm = jnp.sum(x * x, axis=(1, 2))
    xn = x * (1.0 / (nrm + 1.0))[:, None, None, :]
    xn_ref[...] = xn
    p = [xn[:, k // 3, k % 3, :] for k in range(9)]
    I = (p[0] + p[4] + p[8]) * (1.0 / 3.0)
    a01 = 0.5 * (p[1] - p[3])
    a02 = 0.5 * (p[2] - p[6])
    a12 = 0.5 * (p[5] - p[7])
    s00 = p[0] - I
    s11 = p[4] - I
    s01 = 0.5 * (p[1] + p[3])
    s02 = 0.5 * (p[2] + p[6])
    s12 = 0.5 * (p[5] + p[7])
    wt0 = wt0_ref[...]
    wt1 = wt1_ref[...]
    wt2 = wt2_ref[...]
    dot = functools.partial(jnp.dot, preferred_element_type=jnp.float32)
    Iw = dot(I, wt0)
    a01w, a02w, a12w = dot(a01, wt1), dot(a02, wt1), dot(a12, wt1)
    s00w, s01w, s02w = dot(s00, wt2), dot(s01, wt2), dot(s02, wt2)
    s11w, s12w = dot(s11, wt2), dot(s12, wt2)
    for k, v in enumerate([Iw, a01w, a02w, a12w, s00w, s01w, s02w, s11w, s12w]):
        u_ref[k] = v


def _node_prep(Xr, Wt0, Wt1, Wt2):
    full = lambda shp: pl.BlockSpec(shp, lambda i: (0,) * len(shp))
    return pl.pallas_call(
        _node_prep_body,
        grid=(N // BN,),
        in_specs=[
            pl.BlockSpec((BN, 3, 3, 64), lambda i: (i, 0, 0, 0)),
            full((64, 64)), full((64, 64)), full((64, 64)),
        ],
        out_specs=[
            pl.BlockSpec((BN, 3, 3, 64), lambda i: (i, 0, 0, 0)),
            pl.BlockSpec((9, BN, 64), lambda i: (0, i, 0)),
        ],
        out_shape=[
            jax.ShapeDtypeStruct((N, 3, 3, 64), jnp.float32),
            jax.ShapeDtypeStruct((9, N, 64), jnp.float32),
        ],
    )(Xr, Wt0, Wt1, Wt2)


def _edge_mlp_body(ea_ref, ci_ref, cj_ref, ew_ref,
                   w0a_ref, w0b_ref, w0c_ref, b0_ref,
                   w1_ref, b1_ref, w2_ref, b2_ref, xw_ref):
    dot = functools.partial(jnp.dot, preferred_element_type=jnp.float32)
    bf = jnp.bfloat16
    h = (dot(ea_ref[...].astype(bf), w0a_ref[...])
         + dot(ci_ref[...].astype(bf), w0b_ref[...])
         + dot(cj_ref[...].astype(bf), w0c_ref[...]) + b0_ref[...])
    h = jax.nn.silu(h.astype(bf))
    h = jax.nn.silu((dot(h, w1_ref[...]) + b1_ref[...]).astype(bf))
    h = jax.nn.silu((dot(h, w2_ref[...]) + b2_ref[...]).astype(bf))
    d = ew_ref[0]
    c = 0.5 * (jnp.cos(d * (jnp.pi / CUTOFF_UPPER)) + 1.0)
    c = jnp.where(d < CUTOFF_UPPER, c, 0.0)
    h3 = h.astype(jnp.float32).reshape(BE // 128, 128, 192) * c[:, :, None]
    h = h3.reshape(BE, 192)
    xw_ref[0] = h[:, 0:64]
    xw_ref[1] = h[:, 64:128]
    xw_ref[2] = h[:, 128:192]


def _edge_mlp(ea, ci, cj, ew2, W0a, W0b, W0c, b0, W1, b1, W2, b2):
    full = lambda shp: pl.BlockSpec(shp, lambda i: (0,) * len(shp))
    return pl.pallas_call(
        _edge_mlp_body,
        grid=(E // BE,),
        in_specs=[
            pl.BlockSpec((BE, 64), lambda i: (i, 0)),
            pl.BlockSpec((BE, 16), lambda i: (i, 0)),
            pl.BlockSpec((BE, 16), lambda i: (i, 0)),
            pl.BlockSpec((1, BE // 128, 128), lambda i: (i, 0, 0)),
            full((64, 64)), full((16, 64)), full((16, 64)), full((1, 64)),
            full((64, 128)), full((1, 128)), full((128, 192)), full((1, 192)),
        ],
        out_specs=pl.BlockSpec((3, BE, 64), lambda i: (0, i, 0)),
        out_shape=jax.ShapeDtypeStruct((3, E, 64), jnp.float32),
    )(ea, ci, cj, ew2, W0a, W0b, W0c, b0, W1, b1, W2, b2)


def _entries_from_planes(I, a01, a02, a12, s00, s01, s02, s11, s12):
    return [
        I + s00, a01 + s01, a02 + s02,
        -a01 + s01, I + s11, a12 + s12,
        -a02 + s02, -a12 + s12, I - s00 - s11,
    ]


def _epilogue_body(xn_ref, u_ref, m_ref, wt3_ref, wt4_ref, wt5_ref, out_ref):
    u = u_ref[...]
    m = m_ref[...]
    yp = [u[k] for k in range(9)]
    mp = [m[k] + m[9 + k] for k in range(9)]
    y = _entries_from_planes(*yp)
    mm = _entries_from_planes(*mp)
    c = [None] * 9
    for i in range(3):
        for j in range(3):
            acc = 0.0
            for k in range(3):
                acc = acc + y[3 * i + k] * mm[3 * k + j] + mm[3 * i + k] * y[3 * k + j]
            c[3 * i + j] = acc
    I2 = (c[0] + c[4] + c[8]) * (1.0 / 3.0)
    a01 = 0.5 * (c[1] - c[3])
    a02 = 0.5 * (c[2] - c[6])
    a12 = 0.5 * (c[5] - c[7])
    s00 = c[0] - I2
    s11 = c[4] - I2
    s01 = 0.5 * (c[1] + c[3])
    s02 = 0.5 * (c[2] + c[6])
    s12 = 0.5 * (c[5] + c[7])
    nrm = c[0] * c[0]
    for k in range(1, 9):
        nrm = nrm + c[k] * c[k]
    inv = 1.0 / (nrm + 1.0)
    dot = functools.partial(jnp.dot, preferred_element_type=jnp.float32)
    wt3, wt4, wt5 = wt3_ref[...], wt4_ref[...], wt5_ref[...]
    I3 = dot(I2 * inv, wt3)
    a01, a02, a12 = dot(a01 * inv, wt4), dot(a02 * inv, wt4), dot(a12 * inv, wt4)
    s00, s01, s02 = dot(s00 * inv, wt5), dot(s01 * inv, wt5), dot(s02 * inv, wt5)
    s11, s12 = dot(s11 * inv, wt5), dot(s12 * inv, wt5)
    d = _entries_from_planes(I3, a01, a02, a12, s00, s01, s02, s11, s12)
    xn = xn_ref[...]
    outs = []
    for i in range(3):
        for j in range(3):
            dd = 0.0
            for k in range(3):
                dd = dd + d[3 * i + k] * d[3 * k + j]
            outs.append((xn[:, i, j, :] + d[3 * i + j] + dd)[:, None, :])
    out_ref[...] = jnp.concatenate(outs, axis=1).reshape(BN2, 3, 3, 64)


def _epilogue(Xn, U, M, Wt3, Wt4, Wt5):
    full = lambda shp: pl.BlockSpec(shp, lambda i: (0,) * len(shp))
    return pl.pallas_call(
        _epilogue_body,
        grid=(N // BN2,),
        in_specs=[
            pl.BlockSpec((BN2, 3, 3, 64), lambda i: (i, 0, 0, 0)),
            pl.BlockSpec((9, BN2, 64), lambda i: (0, i, 0)),
            pl.BlockSpec((18, BN2, 64), lambda i: (0, i, 0)),
            full((64, 64)), full((64, 64)), full((64, 64)),
        ],
        out_specs=pl.BlockSpec((BN2, 3, 3, 64), lambda i: (i, 0, 0, 0)),
        out_shape=jax.ShapeDtypeStruct((N, 3, 3, 64), jnp.float32),
    )(Xn, U, M, Wt3, Wt4, Wt5)


_GB = 128
_NBATCH = E // _GB
_MESH = plsc.VectorSubcoreMesh(core_axis_name="c", subcore_axis_name="s",
                               num_cores=2, num_subcores=16)


def _gather_charges_body(ch_ref, dst_ref, src_ref, ci_ref, cj_ref,
                         dix, six, rowsa, rowsb, sema, semb):
    c = lax.axis_index("c")
    s = lax.axis_index("s")
    w = s * 2 + c

    def issue(j, slot):
        @pl.when(w + j * 32 < _NBATCH)
        def _():
            off = (w + j * 32) * _GB
            pltpu.sync_copy(dst_ref.at[pl.ds(off, _GB)], dix.at[slot])
            pltpu.sync_copy(src_ref.at[pl.ds(off, _GB)], six.at[slot])
            pltpu.async_copy(ch_ref.at[dix.at[slot]], rowsa.at[slot], sema.at[slot])
            pltpu.async_copy(ch_ref.at[six.at[slot]], rowsb.at[slot], semb.at[slot])

    def finish(j, slot):
        @pl.when(w + j * 32 < _NBATCH)
        def _():
            off = (w + j * 32) * _GB
            pltpu.make_async_copy(ch_ref.at[dix.at[slot]], rowsa.at[slot],
                                  sema.at[slot]).wait()
            pltpu.sync_copy(rowsa.at[slot], ci_ref.at[pl.ds(off, _GB)])
            pltpu.make_async_copy(ch_ref.at[six.at[slot]], rowsb.at[slot],
                                  semb.at[slot]).wait()
            pltpu.sync_copy(rowsb.at[slot], cj_ref.at[pl.ds(off, _GB)])

    issue(0, 0)
    issue(1, 1)

    def body(jj, _):
        for b01 in range(2):
            j = jj * 2 + b01
            finish(j, b01)
            issue(j + 2, b01)
        return 0
    lax.fori_loop(0, (_NBATCH + 63) // 64, body, 0)


def _gather_charges_sc(charges16, dst, src):
    f = pl.kernel(
        _gather_charges_body,
        out_type=[jax.ShapeDtypeStruct((E, 16), jnp.float32),
                  jax.ShapeDtypeStruct((E, 16), jnp.float32)],
        mesh=_MESH,
        compiler_params=pltpu.CompilerParams(use_tc_tiling_on_sc=False),
        scratch_types=[
            pltpu.VMEM((2, _GB), jnp.int32),
            pltpu.VMEM((2, _GB), jnp.int32),
            pltpu.VMEM((2, _GB, 16), jnp.float32),
            pltpu.VMEM((2, _GB, 16), jnp.float32),
            pltpu.SemaphoreType.DMA((2,)),
            pltpu.SemaphoreType.DMA((2,)),
        ],
    )
    return f(charges16, dst, src)


_MB = 128
_EC = E // 2
_MNB = _EC // _MB
_RPT = N // 16
_WSEL = (0, 1, 1, 1, 2, 2, 2, 2, 2)


def _messages_body(u_ref, xw_ref, src_ref, dst_ref, zr_ref, m_ref,
                   acc, usp, ubuf, wbuf, mbuf, sidx, didx,
                   semg, semw, sems, semsi):
    c = lax.axis_index("c")
    t = lax.axis_index("s")

    def make_pass(p):
        w = _WSEL[p]

        def guard(j, fn):
            @pl.when(t + j * 16 < _MNB)
            def _():
                fn()

        def eoff_of(j):
            return c * _EC + (t + j * 16) * _MB

        def stage_i(j, s4):
            def fn():
                pltpu.async_copy(src_ref.at[pl.ds(eoff_of(j), _MB)],
                                 sidx.at[s4], semsi.at[s4])
            guard(j, fn)

        def eoff2_of(j):
            return c * (_EC // 2) + (t + j * 16) * (_MB // 2)

        def stage_g(j, s2, s4):
            def fn():
                pltpu.make_async_copy(src_ref.at[pl.ds(eoff_of(j), _MB)],
                                      sidx.at[s4], semsi.at[s4]).wait()
                pltpu.async_copy(usp.at[sidx.at[s4]], ubuf.at[s2], semg.at[s2])
                pltpu.async_copy(xw_ref.at[w, pl.ds(eoff_of(j), _MB)],
                                 wbuf.at[s2], semw.at[s2])
            guard(j, fn)

        def stage_c(j, s2, s4):
            def fn():
                pltpu.make_async_copy(usp.at[sidx.at[s4]], ubuf.at[s2],
                                      semg.at[s2]).wait()
                pltpu.make_async_copy(xw_ref.at[w, pl.ds(eoff_of(j), _MB)],
                                      wbuf.at[s2], semw.at[s2]).wait()
                pltpu.sync_copy(dst_ref.at[pl.ds(eoff_of(j), _MB)], didx.at[s4])

                @pl.when(j >= 2)
                def _():
                    pltpu.make_async_copy(mbuf.at[s2], acc.at[didx.at[s4]],
                                          sems.at[s2]).wait()

                def edge(e, _):
                    for q in range(4):
                        mbuf[s2, e, pl.ds(q * 16, 16)] = (
                            ubuf[s2, e, pl.ds(q * 16, 16)]
                            * wbuf[s2, e, pl.ds(q * 16, 16)])
                    return 0
                lax.fori_loop(0, _MB, edge, 0)
                pltpu.async_copy(mbuf.at[s2], acc.at[didx.at[s4]],
                                 sems.at[s2], add=True)
            guard(j, fn)

        pltpu.sync_copy(zr_ref, acc.at[pl.ds(t * _RPT, _RPT)])
        pltpu.sync_copy(u_ref.at[p, pl.ds(t * _RPT, _RPT)],
                        usp.at[pl.ds(t * _RPT, _RPT)])
        plsc.subcore_barrier()

        for k in range(4):
            stage_i(k, k)
        stage_g(0, 0, 0)
        stage_g(1, 1, 1)

        def body(jj, _):
            for k in range(4):
                j = jj * 4 + k
                stage_c(j, k % 2, k)
                stage_g(j + 2, k % 2, (k + 2) % 4)
                stage_i(j + 4, k)
            return 0
        lax.fori_loop(0, 10, body, 0)
        for k in range(2):
            j = 38 + k

            @pl.when(t + j * 16 < _MNB)
            def _(j=j, k=k):
                pltpu.make_async_copy(mbuf.at[k], acc.at[didx.at[j % 4]],
                                      sems.at[k]).wait()
        plsc.subcore_barrier()
        for cc in range(2):
            @pl.when(c == cc)
            def _(cc=cc):
                pltpu.sync_copy(acc.at[pl.ds(t * _RPT, _RPT)],
                                m_ref.at[cc * 9 + p, pl.ds(t * _RPT, _RPT)])
        plsc.subcore_barrier()

    for p in range(9):
        make_pass(p)


def _messages_sc(U, xw, src, dst, zrows):
    f = pl.kernel(
        _messages_body,
        out_type=jax.ShapeDtypeStruct((18, N, 64), jnp.float32),
        mesh=_MESH,
        compiler_params=pltpu.CompilerParams(use_tc_tiling_on_sc=False),
        scratch_types=[
            pltpu.VMEM_SHARED((N, 64), jnp.float32),
            pltpu.VMEM_SHARED((N, 64), jnp.float32),
            pltpu.VMEM((2, _MB, 64), jnp.float32),
            pltpu.VMEM((2, _MB, 64), jnp.float32),
            pltpu.VMEM((2, _MB, 64), jnp.float32),
            pltpu.VMEM((4, _MB), jnp.int32),
            pltpu.VMEM((4, _MB), jnp.int32),
            pltpu.SemaphoreType.DMA((2,)),
            pltpu.SemaphoreType.DMA((2,)),
            pltpu.SemaphoreType.DMA((2,)),
            pltpu.SemaphoreType.DMA((4,)),
        ],
    )
    return f(U, xw, src, dst, zrows)


def kernel(X, charges, edge_index, edge_weight, edge_attr,
           W0, b0, W1, b1, W2, b2, Wt0, Wt1, Wt2, Wt3, Wt4, Wt5):
    dst = edge_index[0]
    src = edge_index[1]
    charges16 = jnp.pad(charges, ((0, 0), (0, 8)))
    bf = jnp.bfloat16
    W0a = W0[:64].astype(bf)
    W0b = jnp.pad(W0[64:72], ((0, 8), (0, 0))).astype(bf)
    W0c = jnp.pad(W0[72:80], ((0, 8), (0, 0))).astype(bf)
    ew2 = edge_weight.reshape(E // BE, BE // 128, 128)

    Xn, U = _node_prep(X, Wt0, Wt1, Wt2)
    ci, cj = _gather_charges_sc(charges16, dst, src)
    xw = _edge_mlp(edge_attr, ci, cj, ew2,
                   W0a, W0b, W0c, b0[None, :], W1.astype(bf), b1[None, :],
                   W2.astype(bf), b2[None, :])
    zrows = jnp.zeros((_RPT, 64), jnp.float32)
    M = _messages_sc(U, xw, src, dst, zrows)
    return _epilogue(Xn, U, M, Wt3, Wt4, Wt5)
